# baseline (speedup 1.0000x reference)
import functools

import jax
import jax.numpy as jnp
from jax import lax
from jax.experimental import pallas as pl
from jax.experimental.pallas import tpu as pltpu
from jax.experimental.pallas import tpu_sc as plsc

N = 10000
D = 128
DH = D // 2
E = 320000
NC = 2
NS = 16
NW = NC * NS
CHUNK = 128
NCHUNK = 80
EPW = NCHUNK * CHUNK
ROWS_PAD = 10240
ROWS_PT = ROWS_PAD // NS
ZROWS = ROWS_PT // 4

_mesh = plsc.VectorSubcoreMesh(
    core_axis_name="c", subcore_axis_name="s", num_cores=NC, num_subcores=NS)


def _zero_stage(stage_ref, nrows, ncols):
    z16 = jnp.zeros((16,), jnp.float32)

    def body(i, carry):
        for j in range(ncols // 16):
            stage_ref[i, 16 * j:16 * j + 16] = z16
        return carry

    lax.fori_loop(0, nrows, body, 0)


@functools.partial(
    pl.kernel,
    out_type=jax.ShapeDtypeStruct((NC, ROWS_PAD, 16), jnp.float32),
    mesh=_mesh,
    compiler_params=pltpu.CompilerParams(use_tc_tiling_on_sc=False),
    scratch_types=[
        pltpu.VMEM((NCHUNK, CHUNK), jnp.int32),
        pltpu.VMEM((CHUNK, 16), jnp.float32),
        pltpu.VMEM((ROWS_PT, 16), jnp.float32),
        pltpu.VMEM_SHARED((ROWS_PAD, 16), jnp.float32),
    ],
)
def _sc_degree(dst_hbm, out_hbm, dst_v, ones_v, stage_v, hist_sh):
    c = lax.axis_index("c")
    s = lax.axis_index("s")
    wid = c * NS + s

    one16 = jnp.ones((16,), jnp.float32)

    def fill_ones(i, carry):
        ones_v[i, 0:16] = one16
        return carry

    lax.fori_loop(0, CHUNK, fill_ones, 0)
    _zero_stage(stage_v, ROWS_PT, 16)
    pltpu.sync_copy(stage_v, hist_sh.at[pl.ds(s * ROWS_PT, ROWS_PT), :])
    pltpu.sync_copy(dst_hbm.at[wid], dst_v)
    plsc.subcore_barrier()

    def body(j, carry):
        pltpu.sync_copy(ones_v, hist_sh.at[dst_v.at[j]], add=True)
        return carry

    lax.fori_loop(0, NCHUNK, body, 0)
    plsc.subcore_barrier()

    pltpu.sync_copy(hist_sh.at[pl.ds(s * ROWS_PT, ROWS_PT), :],
                    out_hbm.at[c, pl.ds(s * ROWS_PT, ROWS_PT), :])


NB = 4
LOOKAHEAD = NB // 2


@functools.partial(
    pl.kernel,
    out_type=jax.ShapeDtypeStruct((NC, ROWS_PAD, 2, DH), jnp.float32),
    mesh=_mesh,
    compiler_params=pltpu.CompilerParams(use_tc_tiling_on_sc=False),
    scratch_types=[
        pltpu.VMEM((NCHUNK, CHUNK), jnp.int32),
        pltpu.VMEM((NCHUNK, CHUNK), jnp.int32),
        pltpu.VMEM((NB, CHUNK, DH), jnp.float32),
        pltpu.VMEM((ZROWS, DH), jnp.float32),
        pltpu.VMEM_SHARED((ROWS_PAD, DH), jnp.float32),
        pltpu.SemaphoreType.DMA((NB,)),
        pltpu.SemaphoreType.DMA((NB,)),
    ],
)
def _sc_scatter(y2r_hbm, src_hbm, dst_hbm, out_hbm,
                srcp_v, dst_v, rows_v, stage_v, acc_sh, gsem, ssem):
    c = lax.axis_index("c")
    s = lax.axis_index("s")
    wid = c * NS + s

    _zero_stage(stage_v, ZROWS, DH)
    pltpu.sync_copy(src_hbm.at[wid], srcp_v)
    pltpu.sync_copy(dst_hbm.at[wid], dst_v)

    def gather(j, b):
        pltpu.async_copy(y2r_hbm.at[srcp_v.at[j]], rows_v.at[b], gsem.at[b])

    def gather_wait(b):
        pltpu.make_async_copy(y2r_hbm.at[pl.ds(0, CHUNK), :], rows_v.at[b],
                              gsem.at[b]).wait()

    def scatter(j, b):
        pltpu.async_copy(rows_v.at[b], acc_sh.at[dst_v.at[j]],
                         ssem.at[b], add=True)

    def scatter_wait(b):
        pltpu.make_async_copy(rows_v.at[b], acc_sh.at[pl.ds(0, CHUNK), :],
                              ssem.at[b]).wait()

    for ph in (0, 1):
        def mk_idx(j):
            for k in range(CHUNK // 16):
                sl = slice(16 * k, 16 * k + 16)
                if ph == 0:
                    srcp_v[j, sl] = srcp_v[j, sl] * 2
                else:
                    srcp_v[j, sl] = srcp_v[j, sl] + 1

        for k in range(4):
            pltpu.sync_copy(stage_v,
                            acc_sh.at[pl.ds((s * 4 + k) * ZROWS, ZROWS), :])
        plsc.subcore_barrier()

        for b in range(LOOKAHEAD):
            mk_idx(b)
            gather(b, b)

        def body(i, carry):
            for b in range(NB):
                j = NB * i + b
                bl = (b + LOOKAHEAD) % NB

                @pl.when(j + LOOKAHEAD < NCHUNK)
                def _prefetch():
                    @pl.when(j - LOOKAHEAD >= 0)
                    def _drain():
                        scatter_wait(bl)
                    mk_idx(j + LOOKAHEAD)
                    gather(j + LOOKAHEAD, bl)

                gather_wait(b)
                scatter(j, b)
            return carry

        lax.fori_loop(0, NCHUNK // NB, body, 0)
        for b in range(NB):
            scatter_wait(b)
        plsc.subcore_barrier()

        pltpu.sync_copy(acc_sh.at[pl.ds(s * ROWS_PT, ROWS_PT), :],
                        out_hbm.at[c, pl.ds(s * ROWS_PT, ROWS_PT), ph])
        if ph == 0:
            plsc.subcore_barrier()


_BLK = 2000
_GRID = N // _BLK


def _dinv_block(hist_ref):
    deg = hist_ref[0, :, 0] + hist_ref[1, :, 0] + 1.0
    return lax.rsqrt(deg)


def _combine(p_ref):
    return p_ref[0] + p_ref[1]


def _tc0_body(x_ref, w_ref, xw_ref):
    xw_ref[...] = jnp.dot(x_ref[...], w_ref[...],
                          preferred_element_type=jnp.float32)


def _tc1_body(hist_ref, xw_ref, y_ref):
    y_ref[...] = xw_ref[...] * _dinv_block(hist_ref)[:, None]


def _tc2_body(hist_ref, p_ref, y_ref, w_ref, b_ref, y2_ref):
    dinv = _dinv_block(hist_ref)
    h = (_combine(p_ref) + y_ref[...]) * dinv[:, None] + b_ref[...][None, :]
    h = jnp.maximum(h, 0.0)
    y2_ref[...] = jnp.dot(h, w_ref[...],
                          preferred_element_type=jnp.float32) * dinv[:, None]


def _tc3_body(hist_ref, p_ref, y_ref, b_ref, out_ref):
    dinv = _dinv_block(hist_ref)
    out_ref[...] = (_combine(p_ref) + y_ref[...]) * dinv[:, None] \
        + b_ref[...][None, :]


_hist_spec = pl.BlockSpec((NC, _BLK, 16), lambda i: (0, i, 0))
_rows_spec = pl.BlockSpec((_BLK, D), lambda i: (i, 0))
_part_spec = pl.BlockSpec((NC, _BLK, D), lambda i: (0, i, 0))
_wmat_spec = pl.BlockSpec((D, D), lambda i: (0, 0))
_bias_spec = pl.BlockSpec((D,), lambda i: (0,))
_rows_out = jax.ShapeDtypeStruct((N, D), jnp.float32)


def _tc0(x, w):
    return pl.pallas_call(
        _tc0_body, grid=(_GRID,),
        in_specs=[_rows_spec, _wmat_spec],
        out_specs=_rows_spec, out_shape=_rows_out,
    )(x, w)


def _tc1(hist, xw):
    return pl.pallas_call(
        _tc1_body, grid=(_GRID,),
        in_specs=[_hist_spec, _rows_spec],
        out_specs=_rows_spec, out_shape=_rows_out,
    )(hist, xw)


def _tc2(hist, p, y, w, b):
    return pl.pallas_call(
        _tc2_body, grid=(_GRID,),
        in_specs=[_hist_spec, _part_spec, _rows_spec, _wmat_spec, _bias_spec],
        out_specs=_rows_spec, out_shape=_rows_out,
    )(hist, p, y, w, b)


def _tc3(hist, p, y, b):
    return pl.pallas_call(
        _tc3_body, grid=(_GRID,),
        in_specs=[_hist_spec, _part_spec, _rows_spec, _bias_spec],
        out_specs=_rows_spec, out_shape=_rows_out,
    )(hist, p, y, b)


def kernel(x, edge_index, W1, b1, W2, b2):
    import numpy as np
    ei = edge_index.astype(jnp.int32)
    npad = NW * EPW - E
    pad_src = jnp.asarray(np.arange(npad, dtype=np.int32) % N)
    pad_dst = jnp.asarray(N + np.arange(npad, dtype=np.int32) % (ROWS_PAD - N))
    src3 = jnp.concatenate([ei[0], pad_src]).reshape(NW, NCHUNK, CHUNK)
    dst3 = jnp.concatenate([ei[1], pad_dst]).reshape(NW, NCHUNK, CHUNK)

    def packed(p):
        return p.reshape(NC, ROWS_PAD, D)

    xw1 = _tc0(x, W1)
    hist = _sc_degree(dst3)
    y1 = _tc1(hist, xw1)
    p1 = _sc_scatter(y1.reshape(2 * N, DH), src3, dst3)
    y2 = _tc2(hist, packed(p1), y1, W2, b1)
    p2 = _sc_scatter(y2.reshape(2 * N, DH), src3, dst3)
    return _tc3(hist, packed(p2), y2, b2)


# speedup vs baseline: 1.3192x; 1.3192x over previous
import functools

import jax
import jax.numpy as jnp
from jax import lax
from jax.experimental import pallas as pl
from jax.experimental.pallas import tpu as pltpu
from jax.experimental.pallas import tpu_sc as plsc

N = 10000
D = 128
DH = D // 2
E = 320000
NC = 2
NS = 16
NW = NC * NS
CHUNK = 128
NCHUNK = 80
EPW = NCHUNK * CHUNK
ROWS_PAD = 10240
ROWS_PT = ROWS_PAD // NS
ZROWS = ROWS_PT // 4

_mesh = plsc.VectorSubcoreMesh(
    core_axis_name="c", subcore_axis_name="s", num_cores=NC, num_subcores=NS)


def _zero_stage(stage_ref, nrows, ncols):
    z16 = jnp.zeros((16,), jnp.float32)

    def body(i, carry):
        for j in range(ncols // 16):
            stage_ref[i, 16 * j:16 * j + 16] = z16
        return carry

    lax.fori_loop(0, nrows, body, 0)


@functools.partial(
    pl.kernel,
    out_type=jax.ShapeDtypeStruct((NC, ROWS_PAD, 16), jnp.float32),
    mesh=_mesh,
    compiler_params=pltpu.CompilerParams(use_tc_tiling_on_sc=False),
    scratch_types=[
        pltpu.VMEM((NCHUNK, CHUNK), jnp.int32),
        pltpu.VMEM((CHUNK, 16), jnp.float32),
        pltpu.VMEM((ROWS_PT, 16), jnp.float32),
        pltpu.VMEM_SHARED((ROWS_PAD, 16), jnp.float32),
    ],
)
def _sc_degree(dst_hbm, out_hbm, dst_v, ones_v, stage_v, hist_sh):
    c = lax.axis_index("c")
    s = lax.axis_index("s")
    wid = c * NS + s

    one16 = jnp.ones((16,), jnp.float32)

    def fill_ones(i, carry):
        ones_v[i, 0:16] = one16
        return carry

    lax.fori_loop(0, CHUNK, fill_ones, 0)
    _zero_stage(stage_v, ROWS_PT, 16)
    pltpu.sync_copy(stage_v, hist_sh.at[pl.ds(s * ROWS_PT, ROWS_PT), :])
    pltpu.sync_copy(dst_hbm.at[wid], dst_v)
    plsc.subcore_barrier()

    def body(j, carry):
        pltpu.sync_copy(ones_v, hist_sh.at[dst_v.at[j]], add=True)
        return carry

    lax.fori_loop(0, NCHUNK, body, 0)
    plsc.subcore_barrier()

    pltpu.sync_copy(hist_sh.at[pl.ds(s * ROWS_PT, ROWS_PT), :],
                    out_hbm.at[c, pl.ds(s * ROWS_PT, ROWS_PT), :])


NB = 4
LOOKAHEAD = NB // 2


@functools.partial(
    pl.kernel,
    out_type=jax.ShapeDtypeStruct((NC, 2, ROWS_PAD, DH), jnp.float32),
    mesh=_mesh,
    compiler_params=pltpu.CompilerParams(use_tc_tiling_on_sc=False),
    scratch_types=[
        pltpu.VMEM((NCHUNK, CHUNK), jnp.int32),
        pltpu.VMEM((NCHUNK, CHUNK), jnp.int32),
        pltpu.VMEM((NB, CHUNK, DH), jnp.float32),
        pltpu.VMEM((ZROWS, DH), jnp.float32),
        pltpu.VMEM_SHARED((ROWS_PAD, DH), jnp.float32),
        pltpu.SemaphoreType.DMA((NB,)),
        pltpu.SemaphoreType.DMA((NB,)),
    ],
)
def _sc_scatter(y2r_hbm, src_hbm, dst_hbm, out_hbm,
                srcp_v, dst_v, rows_v, stage_v, acc_sh, gsem, ssem):
    c = lax.axis_index("c")
    s = lax.axis_index("s")
    wid = c * NS + s

    _zero_stage(stage_v, ZROWS, DH)
    pltpu.sync_copy(src_hbm.at[wid], srcp_v)
    pltpu.sync_copy(dst_hbm.at[wid], dst_v)

    def gather(j, b):
        pltpu.async_copy(y2r_hbm.at[srcp_v.at[j]], rows_v.at[b], gsem.at[b])

    def gather_wait(b):
        pltpu.make_async_copy(y2r_hbm.at[pl.ds(0, CHUNK), :], rows_v.at[b],
                              gsem.at[b]).wait()

    def scatter(j, b):
        pltpu.async_copy(rows_v.at[b], acc_sh.at[dst_v.at[j]],
                         ssem.at[b], add=True)

    def scatter_wait(b):
        pltpu.make_async_copy(rows_v.at[b], acc_sh.at[pl.ds(0, CHUNK), :],
                              ssem.at[b]).wait()

    for ph in (0, 1):
        def mk_idx(j):
            for k in range(CHUNK // 16):
                sl = slice(16 * k, 16 * k + 16)
                if ph == 0:
                    srcp_v[j, sl] = srcp_v[j, sl] * 2
                else:
                    srcp_v[j, sl] = srcp_v[j, sl] + 1

        for k in range(4):
            pltpu.sync_copy(stage_v,
                            acc_sh.at[pl.ds((s * 4 + k) * ZROWS, ZROWS), :])
        plsc.subcore_barrier()

        for b in range(LOOKAHEAD):
            mk_idx(b)
            gather(b, b)

        def body(i, carry):
            for b in range(NB):
                j = NB * i + b
                bl = (b + LOOKAHEAD) % NB

                @pl.when(j + LOOKAHEAD < NCHUNK)
                def _prefetch():
                    @pl.when(j - LOOKAHEAD >= 0)
                    def _drain():
                        scatter_wait(bl)
                    mk_idx(j + LOOKAHEAD)
                    gather(j + LOOKAHEAD, bl)

                gather_wait(b)
                scatter(j, b)
            return carry

        lax.fori_loop(0, NCHUNK // NB, body, 0)
        for b in range(NB):
            scatter_wait(b)
        plsc.subcore_barrier()

        pltpu.sync_copy(acc_sh.at[pl.ds(s * ROWS_PT, ROWS_PT), :],
                        out_hbm.at[c, ph, pl.ds(s * ROWS_PT, ROWS_PT), :])
        if ph == 0:
            plsc.subcore_barrier()


_BLK = 2000
_GRID = N // _BLK


def _dinv_block(hist_ref):
    deg = hist_ref[0, :, 0] + hist_ref[1, :, 0] + 1.0
    return lax.rsqrt(deg)


def _combine(p_ref):
    q = p_ref[0] + p_ref[1]
    return jnp.concatenate([q[0], q[1]], axis=-1)


def _tc0_body(x_ref, w_ref, xw_ref):
    xw_ref[...] = jnp.dot(x_ref[...], w_ref[...],
                          preferred_element_type=jnp.float32)


def _tc1_body(hist_ref, xw_ref, y_ref):
    y_ref[...] = xw_ref[...] * _dinv_block(hist_ref)[:, None]


def _tc2_body(hist_ref, p_ref, y_ref, w_ref, b_ref, y2_ref):
    dinv = _dinv_block(hist_ref)
    h = (_combine(p_ref) + y_ref[...]) * dinv[:, None] + b_ref[...][None, :]
    h = jnp.maximum(h, 0.0)
    y2_ref[...] = jnp.dot(h, w_ref[...],
                          preferred_element_type=jnp.float32) * dinv[:, None]


def _tc3_body(hist_ref, p_ref, y_ref, b_ref, out_ref):
    dinv = _dinv_block(hist_ref)
    out_ref[...] = (_combine(p_ref) + y_ref[...]) * dinv[:, None] \
        + b_ref[...][None, :]


_hist_spec = pl.BlockSpec((NC, _BLK, 16), lambda i: (0, i, 0))
_rows_spec = pl.BlockSpec((_BLK, D), lambda i: (i, 0))
_part_spec = pl.BlockSpec((NC, 2, _BLK, DH), lambda i: (0, 0, i, 0))
_wmat_spec = pl.BlockSpec((D, D), lambda i: (0, 0))
_bias_spec = pl.BlockSpec((D,), lambda i: (0,))
_rows_out = jax.ShapeDtypeStruct((N, D), jnp.float32)


def _tc0(x, w):
    return pl.pallas_call(
        _tc0_body, grid=(_GRID,),
        in_specs=[_rows_spec, _wmat_spec],
        out_specs=_rows_spec, out_shape=_rows_out,
    )(x, w)


def _tc1(hist, xw):
    return pl.pallas_call(
        _tc1_body, grid=(_GRID,),
        in_specs=[_hist_spec, _rows_spec],
        out_specs=_rows_spec, out_shape=_rows_out,
    )(hist, xw)


def _tc2(hist, p, y, w, b):
    return pl.pallas_call(
        _tc2_body, grid=(_GRID,),
        in_specs=[_hist_spec, _part_spec, _rows_spec, _wmat_spec, _bias_spec],
        out_specs=_rows_spec, out_shape=_rows_out,
    )(hist, p, y, w, b)


def _tc3(hist, p, y, b):
    return pl.pallas_call(
        _tc3_body, grid=(_GRID,),
        in_specs=[_hist_spec, _part_spec, _rows_spec, _bias_spec],
        out_specs=_rows_spec, out_shape=_rows_out,
    )(hist, p, y, b)


def kernel(x, edge_index, W1, b1, W2, b2):
    import numpy as np
    ei = edge_index.astype(jnp.int32)
    npad = NW * EPW - E
    pad_src = jnp.asarray(np.arange(npad, dtype=np.int32) % N)
    pad_dst = jnp.asarray(N + np.arange(npad, dtype=np.int32) % (ROWS_PAD - N))
    src3 = jnp.concatenate([ei[0], pad_src]).reshape(NW, NCHUNK, CHUNK)
    dst3 = jnp.concatenate([ei[1], pad_dst]).reshape(NW, NCHUNK, CHUNK)

    def packed(p):
        return p

    xw1 = _tc0(x, W1)
    hist = _sc_degree(dst3)
    y1 = _tc1(hist, xw1)
    p1 = _sc_scatter(y1.reshape(2 * N, DH), src3, dst3)
    y2 = _tc2(hist, packed(p1), y1, W2, b1)
    p2 = _sc_scatter(y2.reshape(2 * N, DH), src3, dst3)
    return _tc3(hist, packed(p2), y2, b2)


# speedup vs baseline: 1.4497x; 1.0989x over previous
import functools

import jax
import jax.numpy as jnp
from jax import lax
from jax.experimental import pallas as pl
from jax.experimental.pallas import tpu as pltpu
from jax.experimental.pallas import tpu_sc as plsc

N = 10000
D = 128
DH = D // 2
E = 320000
NC = 2
NS = 16
NW = NC * NS
CHUNK = 128
NCHUNK = 80
EPW = NCHUNK * CHUNK
ROWS_PAD = 10240
ROWS_PT = ROWS_PAD // NS
ZROWS = ROWS_PT // 4

_mesh = plsc.VectorSubcoreMesh(
    core_axis_name="c", subcore_axis_name="s", num_cores=NC, num_subcores=NS)


def _zero_stage(stage_ref, nrows, ncols):
    z16 = jnp.zeros((16,), jnp.float32)

    def body(i, carry):
        for j in range(ncols // 16):
            stage_ref[i, 16 * j:16 * j + 16] = z16
        return carry

    lax.fori_loop(0, nrows, body, 0)


@functools.partial(
    pl.kernel,
    out_type=jax.ShapeDtypeStruct((NC, ROWS_PAD, 16), jnp.float32),
    mesh=_mesh,
    compiler_params=pltpu.CompilerParams(use_tc_tiling_on_sc=False),
    scratch_types=[
        pltpu.VMEM((NCHUNK, CHUNK), jnp.int32),
        pltpu.VMEM((CHUNK, 16), jnp.float32),
        pltpu.VMEM((ROWS_PT, 16), jnp.float32),
        pltpu.VMEM_SHARED((ROWS_PAD, 16), jnp.float32),
    ],
)
def _sc_degree(dst_hbm, out_hbm, dst_v, ones_v, stage_v, hist_sh):
    c = lax.axis_index("c")
    s = lax.axis_index("s")
    wid = c * NS + s

    one16 = jnp.ones((16,), jnp.float32)

    def fill_ones(i, carry):
        ones_v[i, 0:16] = one16
        return carry

    lax.fori_loop(0, CHUNK, fill_ones, 0)
    _zero_stage(stage_v, ROWS_PT, 16)
    pltpu.sync_copy(stage_v, hist_sh.at[pl.ds(s * ROWS_PT, ROWS_PT), :])
    pltpu.sync_copy(dst_hbm.at[wid], dst_v)
    plsc.subcore_barrier()

    def body(j, carry):
        pltpu.sync_copy(ones_v, hist_sh.at[dst_v.at[j]], add=True)
        return carry

    lax.fori_loop(0, NCHUNK, body, 0)
    plsc.subcore_barrier()

    pltpu.sync_copy(hist_sh.at[pl.ds(s * ROWS_PT, ROWS_PT), :],
                    out_hbm.at[c, pl.ds(s * ROWS_PT, ROWS_PT), :])


NB = 4
LOOKAHEAD = NB // 2


@functools.partial(
    pl.kernel,
    out_type=jax.ShapeDtypeStruct((NC, 2, ROWS_PAD, DH), jnp.float32),
    mesh=_mesh,
    compiler_params=pltpu.CompilerParams(use_tc_tiling_on_sc=False),
    scratch_types=[
        pltpu.VMEM((NCHUNK, CHUNK), jnp.int32),
        pltpu.VMEM((NCHUNK, CHUNK), jnp.int32),
        pltpu.VMEM((NB, CHUNK, DH), jnp.float32),
        pltpu.VMEM((ZROWS, DH), jnp.float32),
        pltpu.VMEM_SHARED((ROWS_PAD, DH), jnp.float32),
        pltpu.SemaphoreType.DMA((NB,)),
        pltpu.SemaphoreType.DMA((NB,)),
    ],
)
def _sc_scatter(y2r_hbm, src_hbm, dst_hbm, out_hbm,
                srcp_v, dst_v, rows_v, stage_v, acc_sh, gsem, ssem):
    c = lax.axis_index("c")
    s = lax.axis_index("s")
    wid = c * NS + s

    _zero_stage(stage_v, ZROWS, DH)
    pltpu.sync_copy(src_hbm.at[wid], srcp_v)
    pltpu.sync_copy(dst_hbm.at[wid], dst_v)

    def gather(j, b):
        pltpu.async_copy(y2r_hbm.at[srcp_v.at[j]], rows_v.at[b], gsem.at[b])

    def gather_wait(b):
        pltpu.make_async_copy(y2r_hbm.at[pl.ds(0, CHUNK), :], rows_v.at[b],
                              gsem.at[b]).wait()

    def scatter(j, b):
        pltpu.async_copy(rows_v.at[b], acc_sh.at[dst_v.at[j]],
                         ssem.at[b], add=True)

    def scatter_wait(b):
        pltpu.make_async_copy(rows_v.at[b], acc_sh.at[pl.ds(0, CHUNK), :],
                              ssem.at[b]).wait()

    for ph in (0, 1):
        def mk_idx(j):
            for k in range(CHUNK // 16):
                sl = slice(16 * k, 16 * k + 16)
                if ph == 0:
                    srcp_v[j, sl] = srcp_v[j, sl] * 2
                else:
                    srcp_v[j, sl] = srcp_v[j, sl] + 1

        for k in range(4):
            pltpu.sync_copy(stage_v,
                            acc_sh.at[pl.ds((s * 4 + k) * ZROWS, ZROWS), :])
        plsc.subcore_barrier()

        for b in range(LOOKAHEAD):
            mk_idx(b)
            gather(b, b)

        def body(i, carry):
            for b in range(NB):
                j = NB * i + b
                bl = (b + LOOKAHEAD) % NB

                @pl.when(j + LOOKAHEAD < NCHUNK)
                def _prefetch():
                    @pl.when(j - LOOKAHEAD >= 0)
                    def _drain():
                        scatter_wait(bl)
                    mk_idx(j + LOOKAHEAD)
                    gather(j + LOOKAHEAD, bl)

                gather_wait(b)
                scatter(j, b)
            return carry

        lax.fori_loop(0, NCHUNK // NB, body, 0)
        for b in range(NB):
            scatter_wait(b)
        plsc.subcore_barrier()

        pltpu.sync_copy(acc_sh.at[pl.ds(s * ROWS_PT, ROWS_PT), :],
                        out_hbm.at[c, ph, pl.ds(s * ROWS_PT, ROWS_PT), :])
        if ph == 0:
            plsc.subcore_barrier()


_BLK = 2000
_GRID = N // _BLK


def _dinv_block(hist_ref):
    deg = hist_ref[0, :, 0] + hist_ref[1, :, 0] + 1.0
    return lax.rsqrt(deg)


def _combine(p_ref):
    q = p_ref[0] + p_ref[1]
    ev = jnp.concatenate([q[0, :, :DH], q[1, :, :DH]], axis=1)
    od = jnp.concatenate([q[0, :, DH:], q[1, :, DH:]], axis=1)
    return jnp.stack([ev, od], axis=1).reshape(_BLK, D)


def _tc0_body(x_ref, w_ref, xw_ref):
    xw_ref[...] = jnp.dot(x_ref[...], w_ref[...],
                          preferred_element_type=jnp.float32)


def _tc1_body(hist_ref, xw_ref, y_ref):
    y_ref[...] = xw_ref[...] * _dinv_block(hist_ref)[:, None]


def _tc2_body(hist_ref, p_ref, y_ref, w_ref, b_ref, y2_ref):
    dinv = _dinv_block(hist_ref)
    h = (_combine(p_ref) + y_ref[...]) * dinv[:, None] + b_ref[...][None, :]
    h = jnp.maximum(h, 0.0)
    y2_ref[...] = jnp.dot(h, w_ref[...],
                          preferred_element_type=jnp.float32) * dinv[:, None]


def _tc3_body(hist_ref, p_ref, y_ref, b_ref, out_ref):
    dinv = _dinv_block(hist_ref)
    out_ref[...] = (_combine(p_ref) + y_ref[...]) * dinv[:, None] \
        + b_ref[...][None, :]


_hist_spec = pl.BlockSpec((NC, _BLK, 16), lambda i: (0, i, 0))
_rows_spec = pl.BlockSpec((_BLK, D), lambda i: (i, 0))
_part_spec = pl.BlockSpec((NC, 2, _BLK, DH), lambda i: (0, 0, i, 0))
_wmat_spec = pl.BlockSpec((D, D), lambda i: (0, 0))
_bias_spec = pl.BlockSpec((D,), lambda i: (0,))
_rows_out = jax.ShapeDtypeStruct((N, D), jnp.float32)


def _tc0(x, w):
    return pl.pallas_call(
        _tc0_body, grid=(_GRID,),
        in_specs=[_rows_spec, _wmat_spec],
        out_specs=_rows_spec, out_shape=_rows_out,
    )(x, w)


def _tc1(hist, xw):
    return pl.pallas_call(
        _tc1_body, grid=(_GRID,),
        in_specs=[_hist_spec, _rows_spec],
        out_specs=_rows_spec, out_shape=_rows_out,
    )(hist, xw)


_part_pack = pl.BlockSpec((NC, 2, _BLK // 2, D), lambda i: (0, 0, i, 0))


def _tc2(hist, p, y, w, b):
    return pl.pallas_call(
        _tc2_body, grid=(_GRID,),
        in_specs=[_hist_spec, _part_pack, _rows_spec, _wmat_spec, _bias_spec],
        out_specs=_rows_spec, out_shape=_rows_out,
    )(hist, p, y, w, b)


def _tc3(hist, p, y, b):
    return pl.pallas_call(
        _tc3_body, grid=(_GRID,),
        in_specs=[_hist_spec, _part_pack, _rows_spec, _bias_spec],
        out_specs=_rows_spec, out_shape=_rows_out,
    )(hist, p, y, b)


def kernel(x, edge_index, W1, b1, W2, b2):
    import numpy as np
    ei = edge_index.astype(jnp.int32)
    npad = NW * EPW - E
    pad_src = jnp.asarray(np.arange(npad, dtype=np.int32) % N)
    pad_dst = jnp.asarray(N + np.arange(npad, dtype=np.int32) % (ROWS_PAD - N))
    src3 = jnp.concatenate([ei[0], pad_src]).reshape(NW, NCHUNK, CHUNK)
    dst3 = jnp.concatenate([ei[1], pad_dst]).reshape(NW, NCHUNK, CHUNK)

    def packed(p):
        return p.reshape(NC, 2, ROWS_PAD // 2, D)

    xw1 = _tc0(x, W1)
    hist = _sc_degree(dst3)
    y1 = _tc1(hist, xw1)
    p1 = _sc_scatter(y1.reshape(2 * N, DH), src3, dst3)
    y2 = _tc2(hist, packed(p1), y1, W2, b1)
    p2 = _sc_scatter(y2.reshape(2 * N, DH), src3, dst3)
    return _tc3(hist, packed(p2), y2, b2)


# speedup vs baseline: 1.4717x; 1.0152x over previous
import functools

import jax
import jax.numpy as jnp
from jax import lax
from jax.experimental import pallas as pl
from jax.experimental.pallas import tpu as pltpu
from jax.experimental.pallas import tpu_sc as plsc

N = 10000
D = 128
DH = D // 2
E = 320000
NC = 2
NS = 16
NW = NC * NS
CHUNK = 128
NCHUNK = 80
EPW = NCHUNK * CHUNK
ROWS_PAD = 10240
ROWS_PT = ROWS_PAD // NS
ZROWS = ROWS_PT // 4

_mesh = plsc.VectorSubcoreMesh(
    core_axis_name="c", subcore_axis_name="s", num_cores=NC, num_subcores=NS)


def _zero_stage(stage_ref, nrows, ncols):
    z16 = jnp.zeros((16,), jnp.float32)

    def body(i, carry):
        for j in range(ncols // 16):
            stage_ref[i, 16 * j:16 * j + 16] = z16
        return carry

    lax.fori_loop(0, nrows, body, 0)


@functools.partial(
    pl.kernel,
    out_type=jax.ShapeDtypeStruct((NC, ROWS_PAD, 16), jnp.float32),
    mesh=_mesh,
    compiler_params=pltpu.CompilerParams(use_tc_tiling_on_sc=False),
    scratch_types=[
        pltpu.VMEM((NCHUNK, CHUNK), jnp.int32),
        pltpu.VMEM((CHUNK, 16), jnp.float32),
        pltpu.VMEM((ROWS_PT, 16), jnp.float32),
        pltpu.VMEM_SHARED((ROWS_PAD, 16), jnp.float32),
        pltpu.SemaphoreType.DMA,
    ],
)
def _sc_degree(dst_hbm, out_hbm, dst_v, ones_v, stage_v, hist_sh, hsem):
    c = lax.axis_index("c")
    s = lax.axis_index("s")
    wid = c * NS + s

    one16 = jnp.ones((16,), jnp.float32)

    def fill_ones(i, carry):
        ones_v[i, 0:16] = one16
        return carry

    lax.fori_loop(0, CHUNK, fill_ones, 0)
    _zero_stage(stage_v, ROWS_PT, 16)
    pltpu.sync_copy(stage_v, hist_sh.at[pl.ds(s * ROWS_PT, ROWS_PT), :])
    pltpu.sync_copy(dst_hbm.at[wid], dst_v)
    plsc.subcore_barrier()

    def body(j, carry):
        pltpu.async_copy(ones_v, hist_sh.at[dst_v.at[j]], hsem, add=True)
        return carry

    lax.fori_loop(0, NCHUNK, body, 0)

    def drain(j, carry):
        pltpu.make_async_copy(ones_v, hist_sh.at[pl.ds(0, CHUNK), :],
                              hsem).wait()
        return carry

    lax.fori_loop(0, NCHUNK, drain, 0)
    plsc.subcore_barrier()

    pltpu.sync_copy(hist_sh.at[pl.ds(s * ROWS_PT, ROWS_PT), :],
                    out_hbm.at[c, pl.ds(s * ROWS_PT, ROWS_PT), :])


NB = 4
LOOKAHEAD = NB // 2


@functools.partial(
    pl.kernel,
    out_type=jax.ShapeDtypeStruct((NC, 2, ROWS_PAD, DH), jnp.float32),
    mesh=_mesh,
    compiler_params=pltpu.CompilerParams(use_tc_tiling_on_sc=False),
    scratch_types=[
        pltpu.VMEM((NCHUNK, CHUNK), jnp.int32),
        pltpu.VMEM((NCHUNK, CHUNK), jnp.int32),
        pltpu.VMEM((NB, CHUNK, DH), jnp.float32),
        pltpu.VMEM((ZROWS, DH), jnp.float32),
        pltpu.VMEM_SHARED((ROWS_PAD, DH), jnp.float32),
        pltpu.SemaphoreType.DMA((NB,)),
        pltpu.SemaphoreType.DMA((NB,)),
    ],
)
def _sc_scatter(y2r_hbm, src_hbm, dst_hbm, out_hbm,
                srcp_v, dst_v, rows_v, stage_v, acc_sh, gsem, ssem):
    c = lax.axis_index("c")
    s = lax.axis_index("s")
    wid = c * NS + s

    _zero_stage(stage_v, ZROWS, DH)
    pltpu.sync_copy(src_hbm.at[wid], srcp_v)
    pltpu.sync_copy(dst_hbm.at[wid], dst_v)

    def gather(j, b):
        pltpu.async_copy(y2r_hbm.at[srcp_v.at[j]], rows_v.at[b], gsem.at[b])

    def gather_wait(b):
        pltpu.make_async_copy(y2r_hbm.at[pl.ds(0, CHUNK), :], rows_v.at[b],
                              gsem.at[b]).wait()

    def scatter(j, b):
        pltpu.async_copy(rows_v.at[b], acc_sh.at[dst_v.at[j]],
                         ssem.at[b], add=True)

    def scatter_wait(b):
        pltpu.make_async_copy(rows_v.at[b], acc_sh.at[pl.ds(0, CHUNK), :],
                              ssem.at[b]).wait()

    for ph in (0, 1):
        def mk_idx(j):
            for k in range(CHUNK // 16):
                sl = slice(16 * k, 16 * k + 16)
                if ph == 0:
                    srcp_v[j, sl] = srcp_v[j, sl] * 2
                else:
                    srcp_v[j, sl] = srcp_v[j, sl] + 1

        for k in range(4):
            pltpu.sync_copy(stage_v,
                            acc_sh.at[pl.ds((s * 4 + k) * ZROWS, ZROWS), :])
        plsc.subcore_barrier()

        for b in range(LOOKAHEAD):
            mk_idx(b)
            gather(b, b)

        def body(i, carry):
            for b in range(NB):
                j = NB * i + b
                bl = (b + LOOKAHEAD) % NB

                @pl.when(j + LOOKAHEAD < NCHUNK)
                def _prefetch():
                    @pl.when(j - LOOKAHEAD >= 0)
                    def _drain():
                        scatter_wait(bl)
                    mk_idx(j + LOOKAHEAD)
                    gather(j + LOOKAHEAD, bl)

                gather_wait(b)
                scatter(j, b)
            return carry

        lax.fori_loop(0, NCHUNK // NB, body, 0)
        for b in range(NB):
            scatter_wait(b)
        plsc.subcore_barrier()

        pltpu.sync_copy(acc_sh.at[pl.ds(s * ROWS_PT, ROWS_PT), :],
                        out_hbm.at[c, ph, pl.ds(s * ROWS_PT, ROWS_PT), :])
        if ph == 0:
            plsc.subcore_barrier()


_BLK = 2000
_GRID = N // _BLK


def _dinv_block(hist_ref):
    deg = hist_ref[0, :, 0] + hist_ref[1, :, 0] + 1.0
    return lax.rsqrt(deg)


def _combine(p_ref):
    q = p_ref[0] + p_ref[1]
    ev = jnp.concatenate([q[0, :, :DH], q[1, :, :DH]], axis=1)
    od = jnp.concatenate([q[0, :, DH:], q[1, :, DH:]], axis=1)
    return jnp.stack([ev, od], axis=1).reshape(_BLK, D)


def _tc0_body(x_ref, w_ref, xw_ref):
    xw_ref[...] = jnp.dot(x_ref[...], w_ref[...],
                          preferred_element_type=jnp.float32)


def _tc1_body(hist_ref, xw_ref, y_ref):
    y_ref[...] = xw_ref[...] * _dinv_block(hist_ref)[:, None]


def _tc2_body(hist_ref, p_ref, y_ref, w_ref, b_ref, y2_ref):
    dinv = _dinv_block(hist_ref)
    h = (_combine(p_ref) + y_ref[...]) * dinv[:, None] + b_ref[...][None, :]
    h = jnp.maximum(h, 0.0)
    y2_ref[...] = jnp.dot(h, w_ref[...],
                          preferred_element_type=jnp.float32) * dinv[:, None]


def _tc3_body(hist_ref, p_ref, y_ref, b_ref, out_ref):
    dinv = _dinv_block(hist_ref)
    out_ref[...] = (_combine(p_ref) + y_ref[...]) * dinv[:, None] \
        + b_ref[...][None, :]


_hist_spec = pl.BlockSpec((NC, _BLK, 16), lambda i: (0, i, 0))
_rows_spec = pl.BlockSpec((_BLK, D), lambda i: (i, 0))
_part_spec = pl.BlockSpec((NC, 2, _BLK, DH), lambda i: (0, 0, i, 0))
_wmat_spec = pl.BlockSpec((D, D), lambda i: (0, 0))
_bias_spec = pl.BlockSpec((D,), lambda i: (0,))
_rows_out = jax.ShapeDtypeStruct((N, D), jnp.float32)


def _tc0(x, w):
    return pl.pallas_call(
        _tc0_body, grid=(_GRID,),
        in_specs=[_rows_spec, _wmat_spec],
        out_specs=_rows_spec, out_shape=_rows_out,
    )(x, w)


def _tc1(hist, xw):
    return pl.pallas_call(
        _tc1_body, grid=(_GRID,),
        in_specs=[_hist_spec, _rows_spec],
        out_specs=_rows_spec, out_shape=_rows_out,
    )(hist, xw)


_part_pack = pl.BlockSpec((NC, 2, _BLK // 2, D), lambda i: (0, 0, i, 0))


def _tc2(hist, p, y, w, b):
    return pl.pallas_call(
        _tc2_body, grid=(_GRID,),
        in_specs=[_hist_spec, _part_pack, _rows_spec, _wmat_spec, _bias_spec],
        out_specs=_rows_spec, out_shape=_rows_out,
    )(hist, p, y, w, b)


def _tc3(hist, p, y, b):
    return pl.pallas_call(
        _tc3_body, grid=(_GRID,),
        in_specs=[_hist_spec, _part_pack, _rows_spec, _bias_spec],
        out_specs=_rows_spec, out_shape=_rows_out,
    )(hist, p, y, b)


def kernel(x, edge_index, W1, b1, W2, b2):
    import numpy as np
    ei = edge_index.astype(jnp.int32)
    npad = NW * EPW - E
    pad_src = jnp.asarray(np.arange(npad, dtype=np.int32) % N)
    pad_dst = jnp.asarray(N + np.arange(npad, dtype=np.int32) % (ROWS_PAD - N))
    src3 = jnp.concatenate([ei[0], pad_src]).reshape(NW, NCHUNK, CHUNK)
    dst3 = jnp.concatenate([ei[1], pad_dst]).reshape(NW, NCHUNK, CHUNK)

    def packed(p):
        return p.reshape(NC, 2, ROWS_PAD // 2, D)

    xw1 = _tc0(x, W1)
    hist = _sc_degree(dst3)
    y1 = _tc1(hist, xw1)
    p1 = _sc_scatter(y1.reshape(2 * N, DH), src3, dst3)
    y2 = _tc2(hist, packed(p1), y1, W2, b1)
    p2 = _sc_scatter(y2.reshape(2 * N, DH), src3, dst3)
    return _tc3(hist, packed(p2), y2, b2)
